# baseline (speedup 1.0000x reference)
import functools
import jax
import jax.numpy as jnp
from jax import lax
from jax.experimental import pallas as pl
from jax.experimental.pallas import tpu as pltpu
from jax.experimental.pallas import tpu_sc as plsc

_NC = 2
_NS = 16
_NW = _NC * _NS
_CHUNK = 128


def _mesh():
    return plsc.VectorSubcoreMesh(core_axis_name="c", subcore_axis_name="s")


def _make_deg(n_pad, n_chunks):
    rows = n_pad // _NS

    @functools.partial(
        pl.kernel,
        mesh=_mesh(),
        out_type=jax.ShapeDtypeStruct((_NC * n_pad,), jnp.float32),
        scratch_types=[
            pltpu.VMEM((n_chunks, _CHUNK), jnp.int32),
            pltpu.VMEM((_CHUNK,), jnp.float32),
            pltpu.VMEM_SHARED((n_pad,), jnp.float32),
            pltpu.SemaphoreType.DMA,
        ],
    )
    def deg_kernel(dst_hbm, zeros_hbm, out_hbm, dst_v, ones_v, acc, sem):
        cid = lax.axis_index("c")
        sid = lax.axis_index("s")
        wid = cid * _NS + sid
        pltpu.sync_copy(dst_hbm.at[wid], dst_v)
        for k in range(_CHUNK // 16):
            ones_v[pl.ds(16 * k, 16)] = jnp.full((16,), 1.0, jnp.float32)
        pltpu.sync_copy(zeros_hbm, acc.at[pl.ds(sid * rows, rows)])
        plsc.subcore_barrier()

        def body(j, carry):
            pltpu.sync_copy(ones_v, acc.at[dst_v.at[j]], add=True)
            return carry

        lax.fori_loop(0, n_chunks, body, 0)
        plsc.subcore_barrier()
        pltpu.sync_copy(acc.at[pl.ds(sid * rows, rows)],
                        out_hbm.at[pl.ds(cid * n_pad + sid * rows, rows)])

    return deg_kernel


def _make_propagate(n_pad, d, n_chunks, k):
    rows = n_pad // _NS
    n_t = n_chunks // k

    @functools.partial(
        pl.kernel,
        mesh=_mesh(),
        compiler_params=pltpu.CompilerParams(use_tc_tiling_on_sc=False),
        out_type=jax.ShapeDtypeStruct((_NC, n_pad, d), jnp.float32),
        scratch_types=[
            pltpu.VMEM((n_t, k * _CHUNK), jnp.int32),
            pltpu.VMEM((n_t, k * _CHUNK), jnp.int32),
            pltpu.VMEM((k * _CHUNK, d), jnp.float32),
            pltpu.VMEM_SHARED((n_pad, d), jnp.float32),
            pltpu.SemaphoreType.DMA,
        ],
    )
    def prop_kernel(src_hbm, dst_hbm, table_hbm, zeros_hbm, out_hbm,
                    src_v, dst_v, rows_v, acc, sem):
        cid = lax.axis_index("c")
        sid = lax.axis_index("s")
        wid = cid * _NS + sid
        pltpu.sync_copy(src_hbm.at[wid], src_v)
        pltpu.sync_copy(dst_hbm.at[wid], dst_v)
        pltpu.sync_copy(zeros_hbm, acc.at[pl.ds(sid * rows, rows)])
        plsc.subcore_barrier()

        def body(t, carry):
            pltpu.async_copy(table_hbm.at[src_v.at[t]], rows_v, sem).wait()
            pltpu.sync_copy(rows_v, acc.at[dst_v.at[t]], add=True)
            return carry

        lax.fori_loop(0, n_t, body, 0)
        plsc.subcore_barrier()
        pltpu.sync_copy(acc.at[pl.ds(sid * rows, rows)],
                        out_hbm.at[cid, pl.ds(sid * rows, rows)])

    return prop_kernel


def _pre_body(x_ref, w_ref, degp_ref, dinv_ref, hws_ref):
    deg = degp_ref[0] + degp_ref[1] + 1.0
    dinv = lax.rsqrt(deg)
    dinv_ref[...] = dinv
    hw = jnp.dot(x_ref[...], w_ref[...], preferred_element_type=jnp.float32)
    hws_ref[...] = hw * dinv


def _layer_body(p_ref, hws_ref, dinv_ref, b_ref, w_ref, out_ref):
    dinv = dinv_ref[...]
    agg = (p_ref[0] + p_ref[1] + hws_ref[...]) * dinv + b_ref[...]
    h = jnp.where(agg > 0, agg, 0.2 * agg)
    out_ref[...] = jnp.dot(h, w_ref[...], preferred_element_type=jnp.float32) * dinv


def _final_body(p_ref, hws_ref, dinv_ref, bml_ref, eps_ref, mu_ref, lv_ref, z_ref):
    agg = (p_ref[0] + p_ref[1] + hws_ref[...]) * dinv_ref[...] + bml_ref[...]
    hl = agg.shape[1] // 2
    mu = agg[:, :hl]
    lv = agg[:, hl:]
    mu_ref[...] = mu
    lv_ref[...] = lv
    z_ref[...] = eps_ref[...] * jnp.exp(lv) + mu


def _decoder_body(zr_ref, zc_ref, out_ref):
    prod = lax.dot_general(zr_ref[...], zc_ref[...], (((1,), (1,)), ((), ())),
                           preferred_element_type=jnp.float32)
    out_ref[...] = jax.nn.sigmoid(prod)


def _row_spec(r, width):
    return pl.BlockSpec((r, width), lambda i: (i, 0))


def _full_spec(shape):
    return pl.BlockSpec(shape, lambda i: tuple(0 for _ in shape))


@jax.jit
def kernel(X, A, W1, b1, W2, b2, Wmu, bmu, Wlv, blv, eps):
    N, F_in = X.shape
    E = A.shape[1]
    H1 = W1.shape[1]
    H2 = W2.shape[1]
    L = Wmu.shape[1]

    grain = _NW * _CHUNK * 20
    ep = ((E + grain - 1) // grain) * grain
    n_chunks = ep // (_NW * _CHUNK)
    np_ = ((N + 1 + _NS * 128 - 1) // (_NS * 128)) * (_NS * 128)
    rows = np_ // _NS

    pad = jnp.full((ep - E,), N, jnp.int32)
    src = jnp.concatenate([A[0], pad]).reshape(_NW, n_chunks, _CHUNK)
    dst = jnp.concatenate([A[1], pad]).reshape(_NW, n_chunks, _CHUNK)

    def _by_k(a, k):
        return a.reshape(_NW, n_chunks // k, k * _CHUNK)

    k64, k32 = 1, 20

    zeros1 = jnp.zeros((rows,), jnp.float32)
    zeros_h1 = jnp.zeros((rows, H1), jnp.float32)
    zeros_h2 = jnp.zeros((rows, H2), jnp.float32)
    zeros_ml = jnp.zeros((rows, 2 * L), jnp.float32)

    deg_p = _make_deg(np_, n_chunks)(dst, zeros1).reshape(_NC, np_)

    R = rows
    grid = (np_ // R,)

    degp3 = deg_p[:, :, None]
    dinv, hws1 = pl.pallas_call(
        _pre_body,
        grid=grid,
        in_specs=[
            _row_spec(R, F_in),
            _full_spec((F_in, H1)),
            pl.BlockSpec((_NC, R, 1), lambda i: (0, i, 0)),
        ],
        out_specs=[_row_spec(R, 1), _row_spec(R, H1)],
        out_shape=[
            jax.ShapeDtypeStruct((np_, 1), jnp.float32),
            jax.ShapeDtypeStruct((np_, H1), jnp.float32),
        ],
    )(X, W1, degp3)

    prop1 = _make_propagate(np_, H1, n_chunks, k64)(
        _by_k(src, k64), _by_k(dst, k64), hws1, zeros_h1)

    b1r = b1.reshape(1, H1)
    hws2 = pl.pallas_call(
        _layer_body,
        grid=grid,
        in_specs=[
            pl.BlockSpec((_NC, R, H1), lambda i: (0, i, 0)),
            _row_spec(R, H1),
            _row_spec(R, 1),
            _full_spec((1, H1)),
            _full_spec((H1, H2)),
        ],
        out_specs=_row_spec(R, H2),
        out_shape=jax.ShapeDtypeStruct((np_, H2), jnp.float32),
    )(prop1, hws1, dinv, b1r, W2)

    prop2 = _make_propagate(np_, H2, n_chunks, k32)(
        _by_k(src, k32), _by_k(dst, k32), hws2, zeros_h2)

    b2r = b2.reshape(1, H2)
    Wml = jnp.concatenate([Wmu, Wlv], axis=1)
    hws3 = pl.pallas_call(
        _layer_body,
        grid=grid,
        in_specs=[
            pl.BlockSpec((_NC, R, H2), lambda i: (0, i, 0)),
            _row_spec(R, H2),
            _row_spec(R, 1),
            _full_spec((1, H2)),
            _full_spec((H2, 2 * L)),
        ],
        out_specs=_row_spec(R, 2 * L),
        out_shape=jax.ShapeDtypeStruct((np_, 2 * L), jnp.float32),
    )(prop2, hws2, dinv, b2r, Wml)

    prop3 = _make_propagate(np_, 2 * L, n_chunks, k32)(
        _by_k(src, k32), _by_k(dst, k32), hws3, zeros_ml)

    bml = jnp.concatenate([bmu, blv]).reshape(1, 2 * L)
    mu, logvar, z = pl.pallas_call(
        _final_body,
        grid=grid,
        in_specs=[
            pl.BlockSpec((_NC, R, 2 * L), lambda i: (0, i, 0)),
            _row_spec(R, 2 * L),
            _row_spec(R, 1),
            _full_spec((1, 2 * L)),
            _row_spec(R, L),
        ],
        out_specs=[_row_spec(R, L), _row_spec(R, L), _row_spec(R, L)],
        out_shape=[
            jax.ShapeDtypeStruct((N, L), jnp.float32),
            jax.ShapeDtypeStruct((N, L), jnp.float32),
            jax.ShapeDtypeStruct((N, L), jnp.float32),
        ],
    )(prop3, hws3, dinv, bml, eps)

    BM = BN = 1024
    recon = pl.pallas_call(
        _decoder_body,
        grid=(pl.cdiv(N, BM), pl.cdiv(N, BN)),
        in_specs=[
            pl.BlockSpec((BM, L), lambda i, j: (i, 0)),
            pl.BlockSpec((BN, L), lambda i, j: (j, 0)),
        ],
        out_specs=pl.BlockSpec((BM, BN), lambda i, j: (i, j)),
        out_shape=jax.ShapeDtypeStruct((N, N), jnp.float32),
    )(z, z)

    return (recon, mu, logvar)


# speedup vs baseline: 1.1086x; 1.1086x over previous
import functools
import jax
import jax.numpy as jnp
from jax import lax
from jax.experimental import pallas as pl
from jax.experimental.pallas import tpu as pltpu
from jax.experimental.pallas import tpu_sc as plsc

_NC = 2
_NS = 16
_NW = _NC * _NS
_CHUNK = 128


def _mesh():
    return plsc.VectorSubcoreMesh(core_axis_name="c", subcore_axis_name="s")


def _make_deg(n_pad, n_chunks):
    rows = n_pad // _NS

    @functools.partial(
        pl.kernel,
        mesh=_mesh(),
        out_type=jax.ShapeDtypeStruct((_NC * n_pad,), jnp.float32),
        scratch_types=[
            pltpu.VMEM((n_chunks, _CHUNK), jnp.int32),
            pltpu.VMEM((_CHUNK,), jnp.float32),
            pltpu.VMEM_SHARED((n_pad,), jnp.float32),
            pltpu.SemaphoreType.DMA,
        ],
    )
    def deg_kernel(dst_hbm, zeros_hbm, out_hbm, dst_v, ones_v, acc, sem):
        cid = lax.axis_index("c")
        sid = lax.axis_index("s")
        wid = cid * _NS + sid
        pltpu.sync_copy(dst_hbm.at[wid], dst_v)
        for k in range(_CHUNK // 16):
            ones_v[pl.ds(16 * k, 16)] = jnp.full((16,), 1.0, jnp.float32)
        pltpu.sync_copy(zeros_hbm, acc.at[pl.ds(sid * rows, rows)])
        plsc.subcore_barrier()

        def body(j, carry):
            pltpu.sync_copy(ones_v, acc.at[dst_v.at[j]], add=True)
            return carry

        lax.fori_loop(0, n_chunks, body, 0)
        plsc.subcore_barrier()
        pltpu.sync_copy(acc.at[pl.ds(sid * rows, rows)],
                        out_hbm.at[pl.ds(cid * n_pad + sid * rows, rows)])

    return deg_kernel


def _make_propagate(n_pad, d, n_chunks, k):
    rows = n_pad // _NS
    n_t = n_chunks // k

    @functools.partial(
        pl.kernel,
        mesh=_mesh(),
        compiler_params=pltpu.CompilerParams(use_tc_tiling_on_sc=False),
        out_type=jax.ShapeDtypeStruct((_NC, n_pad, d), jnp.float32),
        scratch_types=[
            pltpu.VMEM((n_t, k * _CHUNK), jnp.int32),
            pltpu.VMEM((n_t, k * _CHUNK), jnp.int32),
            pltpu.VMEM((k * _CHUNK, d), jnp.float32),
            pltpu.VMEM_SHARED((n_pad, d), jnp.float32),
            pltpu.SemaphoreType.DMA,
        ],
    )
    def prop_kernel(src_hbm, dst_hbm, table_hbm, zeros_hbm, out_hbm,
                    src_v, dst_v, rows_v, acc, sem):
        cid = lax.axis_index("c")
        sid = lax.axis_index("s")
        wid = cid * _NS + sid
        pltpu.sync_copy(src_hbm.at[wid], src_v)
        pltpu.sync_copy(dst_hbm.at[wid], dst_v)
        pltpu.sync_copy(zeros_hbm, acc.at[pl.ds(sid * rows, rows)])
        plsc.subcore_barrier()

        def body(t, carry):
            pltpu.async_copy(table_hbm.at[src_v.at[t]], rows_v, sem).wait()
            pltpu.sync_copy(rows_v, acc.at[dst_v.at[t]], add=True)
            return carry

        lax.fori_loop(0, n_t, body, 0)
        plsc.subcore_barrier()
        pltpu.sync_copy(acc.at[pl.ds(sid * rows, rows)],
                        out_hbm.at[cid, pl.ds(sid * rows, rows)])

    return prop_kernel


def _pre_body(x_ref, w_ref, degp_ref, dinv_ref, hws_ref):
    deg = degp_ref[0] + degp_ref[1] + 1.0
    dinv = lax.rsqrt(deg)
    dinv_ref[...] = dinv
    hw = jnp.dot(x_ref[...], w_ref[...], preferred_element_type=jnp.float32)
    hws_ref[...] = hw * dinv


def _layer_body(p_ref, hws_ref, dinv_ref, b_ref, w_ref, out_ref):
    dinv = dinv_ref[...]
    agg = (p_ref[0] + p_ref[1] + hws_ref[...]) * dinv + b_ref[...]
    h = jnp.where(agg > 0, agg, 0.2 * agg)
    out_ref[...] = jnp.dot(h, w_ref[...], preferred_element_type=jnp.float32) * dinv


def _final_body(p_ref, hws_ref, dinv_ref, bml_ref, eps_ref, mu_ref, lv_ref, z_ref):
    agg = (p_ref[0] + p_ref[1] + hws_ref[...]) * dinv_ref[...] + bml_ref[...]
    hl = agg.shape[1] // 2
    mu = agg[:, :hl]
    lv = agg[:, hl:]
    mu_ref[...] = mu
    lv_ref[...] = lv
    z_ref[...] = eps_ref[...] * jnp.exp(lv) + mu


def _decoder_body(zr_ref, zc_ref, out_ref):
    prod = lax.dot_general(zr_ref[...], zc_ref[...], (((1,), (1,)), ((), ())),
                           preferred_element_type=jnp.float32)
    out_ref[...] = jax.nn.sigmoid(prod)


def _row_spec(r, width):
    return pl.BlockSpec((r, width), lambda i: (i, 0))


def _full_spec(shape):
    return pl.BlockSpec(shape, lambda i: tuple(0 for _ in shape))


@jax.jit
def kernel(X, A, W1, b1, W2, b2, Wmu, bmu, Wlv, blv, eps):
    N, F_in = X.shape
    E = A.shape[1]
    H1 = W1.shape[1]
    H2 = W2.shape[1]
    L = Wmu.shape[1]

    grain = _NW * _CHUNK
    ep = ((E + grain - 1) // grain) * grain
    n_chunks = ep // (_NW * _CHUNK)
    np_ = ((N + 1 + _NS * 128 - 1) // (_NS * 128)) * (_NS * 128)
    rows = np_ // _NS

    pad = jnp.full((ep - E,), N, jnp.int32)
    src = jnp.concatenate([A[0], pad]).reshape(_NW, n_chunks, _CHUNK)
    dst = jnp.concatenate([A[1], pad]).reshape(_NW, n_chunks, _CHUNK)

    def _by_k(a, k):
        return a.reshape(_NW, n_chunks // k, k * _CHUNK)

    k64, k32 = 1, 1

    zeros1 = jnp.zeros((rows,), jnp.float32)
    zeros_h1 = jnp.zeros((rows, H1), jnp.float32)
    zeros_h2 = jnp.zeros((rows, H2), jnp.float32)
    zeros_ml = jnp.zeros((rows, 2 * L), jnp.float32)

    deg_p = _make_deg(np_, n_chunks)(dst, zeros1).reshape(_NC, np_)

    R = rows
    grid = (np_ // R,)

    degp3 = deg_p[:, :, None]
    dinv, hws1 = pl.pallas_call(
        _pre_body,
        grid=grid,
        in_specs=[
            _row_spec(R, F_in),
            _full_spec((F_in, H1)),
            pl.BlockSpec((_NC, R, 1), lambda i: (0, i, 0)),
        ],
        out_specs=[_row_spec(R, 1), _row_spec(R, H1)],
        out_shape=[
            jax.ShapeDtypeStruct((np_, 1), jnp.float32),
            jax.ShapeDtypeStruct((np_, H1), jnp.float32),
        ],
    )(X, W1, degp3)

    prop1 = _make_propagate(np_, H1, n_chunks, k64)(
        _by_k(src, k64), _by_k(dst, k64), hws1, zeros_h1)

    b1r = b1.reshape(1, H1)
    hws2 = pl.pallas_call(
        _layer_body,
        grid=grid,
        in_specs=[
            pl.BlockSpec((_NC, R, H1), lambda i: (0, i, 0)),
            _row_spec(R, H1),
            _row_spec(R, 1),
            _full_spec((1, H1)),
            _full_spec((H1, H2)),
        ],
        out_specs=_row_spec(R, H2),
        out_shape=jax.ShapeDtypeStruct((np_, H2), jnp.float32),
    )(prop1, hws1, dinv, b1r, W2)

    prop2 = _make_propagate(np_, H2, n_chunks, k32)(
        _by_k(src, k32), _by_k(dst, k32), hws2, zeros_h2)

    b2r = b2.reshape(1, H2)
    Wml = jnp.concatenate([Wmu, Wlv], axis=1)
    hws3 = pl.pallas_call(
        _layer_body,
        grid=grid,
        in_specs=[
            pl.BlockSpec((_NC, R, H2), lambda i: (0, i, 0)),
            _row_spec(R, H2),
            _row_spec(R, 1),
            _full_spec((1, H2)),
            _full_spec((H2, 2 * L)),
        ],
        out_specs=_row_spec(R, 2 * L),
        out_shape=jax.ShapeDtypeStruct((np_, 2 * L), jnp.float32),
    )(prop2, hws2, dinv, b2r, Wml)

    prop3 = _make_propagate(np_, 2 * L, n_chunks, k32)(
        _by_k(src, k32), _by_k(dst, k32), hws3, zeros_ml)

    bml = jnp.concatenate([bmu, blv]).reshape(1, 2 * L)
    mu, logvar, z = pl.pallas_call(
        _final_body,
        grid=grid,
        in_specs=[
            pl.BlockSpec((_NC, R, 2 * L), lambda i: (0, i, 0)),
            _row_spec(R, 2 * L),
            _row_spec(R, 1),
            _full_spec((1, 2 * L)),
            _row_spec(R, L),
        ],
        out_specs=[_row_spec(R, L), _row_spec(R, L), _row_spec(R, L)],
        out_shape=[
            jax.ShapeDtypeStruct((N, L), jnp.float32),
            jax.ShapeDtypeStruct((N, L), jnp.float32),
            jax.ShapeDtypeStruct((N, L), jnp.float32),
        ],
    )(prop3, hws3, dinv, bml, eps)

    BM = BN = 1024
    recon = pl.pallas_call(
        _decoder_body,
        grid=(pl.cdiv(N, BM), pl.cdiv(N, BN)),
        in_specs=[
            pl.BlockSpec((BM, L), lambda i, j: (i, 0)),
            pl.BlockSpec((BN, L), lambda i, j: (j, 0)),
        ],
        out_specs=pl.BlockSpec((BM, BN), lambda i, j: (i, j)),
        out_shape=jax.ShapeDtypeStruct((N, N), jnp.float32),
    )(z, z)

    return (recon, mu, logvar)


# speedup vs baseline: 1.1124x; 1.0034x over previous
import functools
import jax
import jax.numpy as jnp
from jax import lax
from jax.experimental import pallas as pl
from jax.experimental.pallas import tpu as pltpu
from jax.experimental.pallas import tpu_sc as plsc

_NC = 2
_NS = 16
_NW = _NC * _NS
_CHUNK = 128


def _mesh():
    return plsc.VectorSubcoreMesh(core_axis_name="c", subcore_axis_name="s")


def _make_deg(n_pad, n_chunks):
    rows = n_pad // _NS

    @functools.partial(
        pl.kernel,
        mesh=_mesh(),
        out_type=jax.ShapeDtypeStruct((_NC * n_pad,), jnp.float32),
        scratch_types=[
            pltpu.VMEM((n_chunks, _CHUNK), jnp.int32),
            pltpu.VMEM((_CHUNK,), jnp.float32),
            pltpu.VMEM_SHARED((n_pad,), jnp.float32),
            pltpu.SemaphoreType.DMA,
        ],
    )
    def deg_kernel(dst_hbm, zeros_hbm, out_hbm, dst_v, ones_v, acc, sem):
        cid = lax.axis_index("c")
        sid = lax.axis_index("s")
        wid = cid * _NS + sid
        pltpu.sync_copy(dst_hbm.at[wid], dst_v)
        for k in range(_CHUNK // 16):
            ones_v[pl.ds(16 * k, 16)] = jnp.full((16,), 1.0, jnp.float32)
        pltpu.sync_copy(zeros_hbm, acc.at[pl.ds(sid * rows, rows)])
        plsc.subcore_barrier()

        def body(j, carry):
            pltpu.sync_copy(ones_v, acc.at[dst_v.at[j]], add=True)
            return carry

        lax.fori_loop(0, n_chunks, body, 0)
        plsc.subcore_barrier()
        pltpu.sync_copy(acc.at[pl.ds(sid * rows, rows)],
                        out_hbm.at[pl.ds(cid * n_pad + sid * rows, rows)])

    return deg_kernel


def _make_propagate(n_pad, d, n_chunks, k):
    rows = n_pad // _NS
    n_t = n_chunks // k

    @functools.partial(
        pl.kernel,
        mesh=_mesh(),
        compiler_params=pltpu.CompilerParams(use_tc_tiling_on_sc=False),
        out_type=jax.ShapeDtypeStruct((_NC, n_pad, d), jnp.float32),
        scratch_types=[
            pltpu.VMEM((n_t, k * _CHUNK), jnp.int32),
            pltpu.VMEM((n_t, k * _CHUNK), jnp.int32),
            pltpu.VMEM((k * _CHUNK, d), jnp.float32),
            pltpu.VMEM_SHARED((n_pad, d), jnp.float32),
            pltpu.SemaphoreType.DMA,
        ],
    )
    def prop_kernel(src_hbm, dst_hbm, table_hbm, zeros_hbm, out_hbm,
                    src_v, dst_v, rows_v, acc, sem):
        cid = lax.axis_index("c")
        sid = lax.axis_index("s")
        wid = cid * _NS + sid
        pltpu.sync_copy(src_hbm.at[wid], src_v)
        pltpu.sync_copy(dst_hbm.at[wid], dst_v)
        pltpu.sync_copy(zeros_hbm, acc.at[pl.ds(sid * rows, rows)])
        plsc.subcore_barrier()

        def body(t, carry):
            pltpu.async_copy(table_hbm.at[src_v.at[t]], rows_v, sem).wait()
            pltpu.sync_copy(rows_v, acc.at[dst_v.at[t]], add=True)
            return carry

        lax.fori_loop(0, n_t, body, 0)
        plsc.subcore_barrier()
        pltpu.sync_copy(acc.at[pl.ds(sid * rows, rows)],
                        out_hbm.at[cid, pl.ds(sid * rows, rows)])

    return prop_kernel


def _pre_body(x_ref, w_ref, degp_ref, dinv_ref, hws_ref):
    deg = degp_ref[0] + degp_ref[1] + 1.0
    dinv = lax.rsqrt(deg)
    dinv_ref[...] = dinv
    hw = jnp.dot(x_ref[...], w_ref[...], preferred_element_type=jnp.float32)
    hws_ref[...] = hw * dinv


def _layer_body(p_ref, hws_ref, dinv_ref, b_ref, w_ref, out_ref):
    dinv = dinv_ref[...]
    agg = (p_ref[0] + p_ref[1] + hws_ref[...]) * dinv + b_ref[...]
    h = jnp.where(agg > 0, agg, 0.2 * agg)
    out_ref[...] = jnp.dot(h, w_ref[...], preferred_element_type=jnp.float32) * dinv


def _final_body(p_ref, hws_ref, dinv_ref, bml_ref, eps_ref, mu_ref, lv_ref, z_ref):
    agg = (p_ref[0] + p_ref[1] + hws_ref[...]) * dinv_ref[...] + bml_ref[...]
    hl = agg.shape[1] // 2
    mu = agg[:, :hl]
    lv = agg[:, hl:]
    mu_ref[...] = mu
    lv_ref[...] = lv
    z_ref[...] = eps_ref[...] * jnp.exp(lv) + mu


def _decoder_body(zr_ref, zc_ref, out_ref):
    prod = lax.dot_general(zr_ref[...], zc_ref[...], (((1,), (1,)), ((), ())),
                           preferred_element_type=jnp.float32)
    out_ref[...] = jax.nn.sigmoid(prod)


def _row_spec(r, width):
    return pl.BlockSpec((r, width), lambda i: (i, 0))


def _full_spec(shape):
    return pl.BlockSpec(shape, lambda i: tuple(0 for _ in shape))


@jax.jit
def kernel(X, A, W1, b1, W2, b2, Wmu, bmu, Wlv, blv, eps):
    N, F_in = X.shape
    E = A.shape[1]
    H1 = W1.shape[1]
    H2 = W2.shape[1]
    L = Wmu.shape[1]

    grain = _NW * _CHUNK
    ep = ((E + grain - 1) // grain) * grain
    n_chunks = ep // (_NW * _CHUNK)
    np_ = ((N + 1 + _NS * 128 - 1) // (_NS * 128)) * (_NS * 128)
    rows = np_ // _NS

    pad = jnp.full((ep - E,), N, jnp.int32)
    src = jnp.concatenate([A[0], pad]).reshape(_NW, n_chunks, _CHUNK)
    dst = jnp.concatenate([A[1], pad]).reshape(_NW, n_chunks, _CHUNK)

    def _by_k(a, k):
        return a.reshape(_NW, n_chunks // k, k * _CHUNK)

    k64, k32 = 1, 1

    zeros1 = jnp.zeros((rows,), jnp.float32)
    zeros_h1 = jnp.zeros((rows, H1), jnp.float32)
    zeros_h2 = jnp.zeros((rows, H2), jnp.float32)
    zeros_ml = jnp.zeros((rows, 2 * L), jnp.float32)

    deg_p = _make_deg(np_, n_chunks)(dst, zeros1).reshape(_NC, np_)

    R = 2560
    grid = (np_ // R,)

    degp3 = deg_p[:, :, None]
    dinv, hws1 = pl.pallas_call(
        _pre_body,
        grid=grid,
        in_specs=[
            _row_spec(R, F_in),
            _full_spec((F_in, H1)),
            pl.BlockSpec((_NC, R, 1), lambda i: (0, i, 0)),
        ],
        out_specs=[_row_spec(R, 1), _row_spec(R, H1)],
        out_shape=[
            jax.ShapeDtypeStruct((np_, 1), jnp.float32),
            jax.ShapeDtypeStruct((np_, H1), jnp.float32),
        ],
    )(X, W1, degp3)

    prop1 = _make_propagate(np_, H1, n_chunks, k64)(
        _by_k(src, k64), _by_k(dst, k64), hws1, zeros_h1)

    b1r = b1.reshape(1, H1)
    hws2 = pl.pallas_call(
        _layer_body,
        grid=grid,
        in_specs=[
            pl.BlockSpec((_NC, R, H1), lambda i: (0, i, 0)),
            _row_spec(R, H1),
            _row_spec(R, 1),
            _full_spec((1, H1)),
            _full_spec((H1, H2)),
        ],
        out_specs=_row_spec(R, H2),
        out_shape=jax.ShapeDtypeStruct((np_, H2), jnp.float32),
    )(prop1, hws1, dinv, b1r, W2)

    prop2 = _make_propagate(np_, H2, n_chunks, k32)(
        _by_k(src, k32), _by_k(dst, k32), hws2, zeros_h2)

    b2r = b2.reshape(1, H2)
    Wml = jnp.concatenate([Wmu, Wlv], axis=1)
    hws3 = pl.pallas_call(
        _layer_body,
        grid=grid,
        in_specs=[
            pl.BlockSpec((_NC, R, H2), lambda i: (0, i, 0)),
            _row_spec(R, H2),
            _row_spec(R, 1),
            _full_spec((1, H2)),
            _full_spec((H2, 2 * L)),
        ],
        out_specs=_row_spec(R, 2 * L),
        out_shape=jax.ShapeDtypeStruct((np_, 2 * L), jnp.float32),
    )(prop2, hws2, dinv, b2r, Wml)

    prop3 = _make_propagate(np_, 2 * L, n_chunks, k32)(
        _by_k(src, k32), _by_k(dst, k32), hws3, zeros_ml)

    bml = jnp.concatenate([bmu, blv]).reshape(1, 2 * L)
    mu, logvar, z = pl.pallas_call(
        _final_body,
        grid=grid,
        in_specs=[
            pl.BlockSpec((_NC, R, 2 * L), lambda i: (0, i, 0)),
            _row_spec(R, 2 * L),
            _row_spec(R, 1),
            _full_spec((1, 2 * L)),
            _row_spec(R, L),
        ],
        out_specs=[_row_spec(R, L), _row_spec(R, L), _row_spec(R, L)],
        out_shape=[
            jax.ShapeDtypeStruct((N, L), jnp.float32),
            jax.ShapeDtypeStruct((N, L), jnp.float32),
            jax.ShapeDtypeStruct((N, L), jnp.float32),
        ],
    )(prop3, hws3, dinv, bml, eps)

    BM, BN = 1024, 2048
    recon = pl.pallas_call(
        _decoder_body,
        grid=(pl.cdiv(N, BM), pl.cdiv(N, BN)),
        in_specs=[
            pl.BlockSpec((BM, L), lambda i, j: (i, 0)),
            pl.BlockSpec((BN, L), lambda i, j: (j, 0)),
        ],
        out_specs=pl.BlockSpec((BM, BN), lambda i, j: (i, j)),
        out_shape=jax.ShapeDtypeStruct((N, N), jnp.float32),
    )(z, z)

    return (recon, mu, logvar)


# speedup vs baseline: 1.1125x; 1.0000x over previous
import functools
import jax
import jax.numpy as jnp
from jax import lax
from jax.experimental import pallas as pl
from jax.experimental.pallas import tpu as pltpu
from jax.experimental.pallas import tpu_sc as plsc

_NC = 2
_NS = 16
_NW = _NC * _NS
_CHUNK = 128


def _mesh():
    return plsc.VectorSubcoreMesh(core_axis_name="c", subcore_axis_name="s")


def _make_deg(n_pad, n_chunks):
    rows = n_pad // _NS

    @functools.partial(
        pl.kernel,
        mesh=_mesh(),
        out_type=jax.ShapeDtypeStruct((_NC * n_pad,), jnp.float32),
        scratch_types=[
            pltpu.VMEM((n_chunks, _CHUNK), jnp.int32),
            pltpu.VMEM((_CHUNK,), jnp.float32),
            pltpu.VMEM_SHARED((n_pad,), jnp.float32),
            pltpu.SemaphoreType.DMA,
        ],
    )
    def deg_kernel(dst_hbm, zeros_hbm, out_hbm, dst_v, ones_v, acc, sem):
        cid = lax.axis_index("c")
        sid = lax.axis_index("s")
        wid = cid * _NS + sid
        pltpu.sync_copy(dst_hbm.at[wid], dst_v)
        for k in range(_CHUNK // 16):
            ones_v[pl.ds(16 * k, 16)] = jnp.full((16,), 1.0, jnp.float32)
        pltpu.sync_copy(zeros_hbm, acc.at[pl.ds(sid * rows, rows)])
        plsc.subcore_barrier()

        def body(j, carry):
            pltpu.sync_copy(ones_v, acc.at[dst_v.at[j]], add=True)
            return carry

        lax.fori_loop(0, n_chunks, body, 0)
        plsc.subcore_barrier()
        pltpu.sync_copy(acc.at[pl.ds(sid * rows, rows)],
                        out_hbm.at[pl.ds(cid * n_pad + sid * rows, rows)])

    return deg_kernel


def _make_propagate(n_pad, d, n_chunks, k):
    rows = n_pad // _NS
    n_t = n_chunks // k

    @functools.partial(
        pl.kernel,
        mesh=_mesh(),
        compiler_params=pltpu.CompilerParams(use_tc_tiling_on_sc=False),
        out_type=jax.ShapeDtypeStruct((_NC, n_pad, d), jnp.float32),
        scratch_types=[
            pltpu.VMEM((n_t, k * _CHUNK), jnp.int32),
            pltpu.VMEM((n_t, k * _CHUNK), jnp.int32),
            pltpu.VMEM((k * _CHUNK, d), jnp.float32),
            pltpu.VMEM((_CHUNK, d), jnp.float32),
            pltpu.VMEM_SHARED((n_pad, d), jnp.float32),
            pltpu.SemaphoreType.DMA,
        ],
    )
    def prop_kernel(src_hbm, dst_hbm, table_hbm, out_hbm,
                    src_v, dst_v, rows_v, zbuf, acc, sem):
        cid = lax.axis_index("c")
        sid = lax.axis_index("s")
        wid = cid * _NS + sid
        pltpu.sync_copy(src_hbm.at[wid], src_v)
        pltpu.sync_copy(dst_hbm.at[wid], dst_v)

        zv = jnp.zeros((16,), jnp.float32)

        def zrow(r, carry):
            for c in range(d // 16):
                zbuf[r, pl.ds(16 * c, 16)] = zv
            return carry

        lax.fori_loop(0, _CHUNK, zrow, 0)
        for z in range(rows // _CHUNK):
            pltpu.sync_copy(zbuf, acc.at[pl.ds(sid * rows + z * _CHUNK, _CHUNK)])
        plsc.subcore_barrier()

        def body(t, carry):
            pltpu.async_copy(table_hbm.at[src_v.at[t]], rows_v, sem).wait()
            pltpu.sync_copy(rows_v, acc.at[dst_v.at[t]], add=True)
            return carry

        lax.fori_loop(0, n_t, body, 0)
        plsc.subcore_barrier()
        pltpu.sync_copy(acc.at[pl.ds(sid * rows, rows)],
                        out_hbm.at[cid, pl.ds(sid * rows, rows)])

    return prop_kernel


def _pre_body(x_ref, w_ref, degp_ref, dinv_ref, hws_ref):
    deg = degp_ref[0] + degp_ref[1] + 1.0
    dinv = lax.rsqrt(deg)
    dinv_ref[...] = dinv
    hw = jnp.dot(x_ref[...], w_ref[...], preferred_element_type=jnp.float32)
    hws_ref[...] = hw * dinv


def _layer_body(p_ref, hws_ref, dinv_ref, b_ref, w_ref, out_ref):
    dinv = dinv_ref[...]
    agg = (p_ref[0] + p_ref[1] + hws_ref[...]) * dinv + b_ref[...]
    h = jnp.where(agg > 0, agg, 0.2 * agg)
    out_ref[...] = jnp.dot(h, w_ref[...], preferred_element_type=jnp.float32) * dinv


def _final_body(p_ref, hws_ref, dinv_ref, bml_ref, eps_ref, mu_ref, lv_ref, z_ref):
    agg = (p_ref[0] + p_ref[1] + hws_ref[...]) * dinv_ref[...] + bml_ref[...]
    hl = agg.shape[1] // 2
    mu = agg[:, :hl]
    lv = agg[:, hl:]
    mu_ref[...] = mu
    lv_ref[...] = lv
    z_ref[...] = eps_ref[...] * jnp.exp(lv) + mu


def _decoder_body(zr_ref, zc_ref, out_ref):
    prod = lax.dot_general(zr_ref[...], zc_ref[...], (((1,), (1,)), ((), ())),
                           preferred_element_type=jnp.float32)
    out_ref[...] = jax.nn.sigmoid(prod)


def _row_spec(r, width):
    return pl.BlockSpec((r, width), lambda i: (i, 0))


def _full_spec(shape):
    return pl.BlockSpec(shape, lambda i: tuple(0 for _ in shape))


@jax.jit
def kernel(X, A, W1, b1, W2, b2, Wmu, bmu, Wlv, blv, eps):
    N, F_in = X.shape
    E = A.shape[1]
    H1 = W1.shape[1]
    H2 = W2.shape[1]
    L = Wmu.shape[1]

    grain = _NW * _CHUNK * 4
    ep = ((E + grain - 1) // grain) * grain
    n_chunks = ep // (_NW * _CHUNK)
    np_ = ((N + 1 + _NS * 128 - 1) // (_NS * 128)) * (_NS * 128)
    rows = np_ // _NS

    pad = jnp.full((ep - E,), N, jnp.int32)
    src = jnp.concatenate([A[0], pad]).reshape(_NW, n_chunks, _CHUNK)
    dst = jnp.concatenate([A[1], pad]).reshape(_NW, n_chunks, _CHUNK)

    def _by_k(a, k):
        return a.reshape(_NW, n_chunks // k, k * _CHUNK)

    k64, k32 = 4, 20

    zeros1 = jnp.zeros((rows,), jnp.float32)

    deg_p = _make_deg(np_, n_chunks)(dst, zeros1).reshape(_NC, np_)

    R = 2560
    grid = (np_ // R,)

    degp3 = deg_p[:, :, None]
    dinv, hws1 = pl.pallas_call(
        _pre_body,
        grid=grid,
        in_specs=[
            _row_spec(R, F_in),
            _full_spec((F_in, H1)),
            pl.BlockSpec((_NC, R, 1), lambda i: (0, i, 0)),
        ],
        out_specs=[_row_spec(R, 1), _row_spec(R, H1)],
        out_shape=[
            jax.ShapeDtypeStruct((np_, 1), jnp.float32),
            jax.ShapeDtypeStruct((np_, H1), jnp.float32),
        ],
    )(X, W1, degp3)

    prop1 = _make_propagate(np_, H1, n_chunks, k64)(
        _by_k(src, k64), _by_k(dst, k64), hws1)

    b1r = b1.reshape(1, H1)
    hws2 = pl.pallas_call(
        _layer_body,
        grid=grid,
        in_specs=[
            pl.BlockSpec((_NC, R, H1), lambda i: (0, i, 0)),
            _row_spec(R, H1),
            _row_spec(R, 1),
            _full_spec((1, H1)),
            _full_spec((H1, H2)),
        ],
        out_specs=_row_spec(R, H2),
        out_shape=jax.ShapeDtypeStruct((np_, H2), jnp.float32),
    )(prop1, hws1, dinv, b1r, W2)

    prop2 = _make_propagate(np_, H2, n_chunks, k32)(
        _by_k(src, k32), _by_k(dst, k32), hws2)

    b2r = b2.reshape(1, H2)
    Wml = jnp.concatenate([Wmu, Wlv], axis=1)
    hws3 = pl.pallas_call(
        _layer_body,
        grid=grid,
        in_specs=[
            pl.BlockSpec((_NC, R, H2), lambda i: (0, i, 0)),
            _row_spec(R, H2),
            _row_spec(R, 1),
            _full_spec((1, H2)),
            _full_spec((H2, 2 * L)),
        ],
        out_specs=_row_spec(R, 2 * L),
        out_shape=jax.ShapeDtypeStruct((np_, 2 * L), jnp.float32),
    )(prop2, hws2, dinv, b2r, Wml)

    prop3 = _make_propagate(np_, 2 * L, n_chunks, k32)(
        _by_k(src, k32), _by_k(dst, k32), hws3)

    bml = jnp.concatenate([bmu, blv]).reshape(1, 2 * L)
    mu, logvar, z = pl.pallas_call(
        _final_body,
        grid=grid,
        in_specs=[
            pl.BlockSpec((_NC, R, 2 * L), lambda i: (0, i, 0)),
            _row_spec(R, 2 * L),
            _row_spec(R, 1),
            _full_spec((1, 2 * L)),
            _row_spec(R, L),
        ],
        out_specs=[_row_spec(R, L), _row_spec(R, L), _row_spec(R, L)],
        out_shape=[
            jax.ShapeDtypeStruct((N, L), jnp.float32),
            jax.ShapeDtypeStruct((N, L), jnp.float32),
            jax.ShapeDtypeStruct((N, L), jnp.float32),
        ],
    )(prop3, hws3, dinv, bml, eps)

    BM, BN = 1024, 2048
    recon = pl.pallas_call(
        _decoder_body,
        grid=(pl.cdiv(N, BM), pl.cdiv(N, BN)),
        in_specs=[
            pl.BlockSpec((BM, L), lambda i, j: (i, 0)),
            pl.BlockSpec((BN, L), lambda i, j: (j, 0)),
        ],
        out_specs=pl.BlockSpec((BM, BN), lambda i, j: (i, j)),
        out_shape=jax.ShapeDtypeStruct((N, N), jnp.float32),
    )(z, z)

    return (recon, mu, logvar)
